# Initial kernel scaffold; baseline (speedup 1.0000x reference)
import jax
import jax.numpy as jnp
from jax.experimental import pallas as pl


def kernel(text_emb, img_emb, v_missing, entity_residual, residual_scale, rel_emb_fusion, Wg, bg, gamma, beta, rel_emb_dec, has_img, pos_triples, neg_triples):
    raise NotImplementedError("write your pallas kernel here")


# speedup vs baseline: 1.8058x; 1.8058x over previous
import functools

import jax
import jax.numpy as jnp
from jax import lax
from jax.experimental import pallas as pl
from jax.experimental.pallas import tpu as pltpu
from jax.experimental.pallas import tpu_sc as plsc

N_ENT = 100000
N_REL = 1000
D = 128
B_POS = 16384
B_NEG = 65536
B_ALL = B_POS + B_NEG
N_EID = 2 * B_ALL

NC = 2
NS = 16
NW = NC * NS
CHUNK = 128

EID_PER_W = N_EID // NW
RID_PER_W = B_ALL // NW
EID_CHUNKS = EID_PER_W // CHUNK
RID_CHUNKS = RID_PER_W // CHUNK

ROWS_B = 512
N_BLK = B_ALL // ROWS_B
POS_BLKS = B_POS // ROWS_B


def _sc_gather(text_emb, img_emb, entity_residual, has_img_f,
               rel_emb_fusion, rel_emb_dec, eids, rids):
    mesh = plsc.VectorSubcoreMesh(core_axis_name="c", subcore_axis_name="s")

    @functools.partial(
        pl.kernel,
        mesh=mesh,
        out_type=[
            jax.ShapeDtypeStruct((N_EID, D), jnp.float32),
            jax.ShapeDtypeStruct((N_EID, D), jnp.float32),
            jax.ShapeDtypeStruct((N_EID, D), jnp.float32),
            jax.ShapeDtypeStruct((N_EID,), jnp.float32),
            jax.ShapeDtypeStruct((B_ALL, D), jnp.float32),
            jax.ShapeDtypeStruct((B_ALL, D), jnp.float32),
        ],
        scratch_types=[
            pltpu.VMEM((CHUNK,), jnp.int32),
            pltpu.VMEM((CHUNK, D), jnp.float32),
            pltpu.VMEM((CHUNK,), jnp.float32),
            pltpu.SemaphoreType.DMA,
        ],
    )
    def k(text_h, img_h, res_h, mask_h, rf_h, rd_h, eids_h, rids_h,
          out_t, out_v, out_r, out_m, out_rf, out_rd,
          idx_v, rows_v, mrow_v, sem):
        wid = lax.axis_index("c") * NS + lax.axis_index("s")
        ebase = wid * EID_PER_W
        rbase = wid * RID_PER_W

        def ebody(c, _):
            off = ebase + c * CHUNK
            pltpu.sync_copy(eids_h.at[pl.ds(off, CHUNK)], idx_v)
            pltpu.async_copy(text_h.at[idx_v], rows_v, sem).wait()
            pltpu.sync_copy(rows_v, out_t.at[pl.ds(off, CHUNK)])
            pltpu.async_copy(img_h.at[idx_v], rows_v, sem).wait()
            pltpu.sync_copy(rows_v, out_v.at[pl.ds(off, CHUNK)])
            pltpu.async_copy(res_h.at[idx_v], rows_v, sem).wait()
            pltpu.sync_copy(rows_v, out_r.at[pl.ds(off, CHUNK)])
            pltpu.async_copy(mask_h.at[idx_v], mrow_v, sem).wait()
            pltpu.sync_copy(mrow_v, out_m.at[pl.ds(off, CHUNK)])
            return 0

        lax.fori_loop(0, EID_CHUNKS, ebody, 0)

        def rbody(c, _):
            off = rbase + c * CHUNK
            pltpu.sync_copy(rids_h.at[pl.ds(off, CHUNK)], idx_v)
            pltpu.async_copy(rf_h.at[idx_v], rows_v, sem).wait()
            pltpu.sync_copy(rows_v, out_rf.at[pl.ds(off, CHUNK)])
            pltpu.async_copy(rd_h.at[idx_v], rows_v, sem).wait()
            pltpu.sync_copy(rows_v, out_rd.at[pl.ds(off, CHUNK)])
            return 0

        lax.fori_loop(0, RID_CHUNKS, rbody, 0)

    return k(text_emb, img_emb, entity_residual, has_img_f,
             rel_emb_fusion, rel_emb_dec, eids, rids)


def _softplus(x):
    return jnp.maximum(x, 0.0) + jnp.log1p(jnp.exp(-jnp.abs(x)))


def _fuse_side(t, v_raw, m, resid, rf, Wg, bg, gamma, beta, scale, v_missing):
    v = jnp.where(m > 0.5, v_raw, v_missing)
    x = jnp.concatenate([t, v, rf], axis=1)
    g = jax.nn.sigmoid(
        jnp.dot(x, Wg, preferred_element_type=jnp.float32) + bg)
    z = g * t + (1.0 - g) * v
    mu = jnp.mean(z, axis=-1, keepdims=True)
    zc = z - mu
    var = jnp.mean(zc * zc, axis=-1, keepdims=True)
    z = zc * jax.lax.rsqrt(var + 1e-05) * gamma + beta
    return z + scale * resid


def _fuse_kernel(th_ref, tt_ref, vh_ref, vt_ref, mh_ref, mt_ref,
                 rh_ref, rt_ref, rf_ref, rd_ref,
                 wg_ref, bg_ref, gm_ref, bt_ref, vm_ref, rs_ref,
                 out_ref):
    i = pl.program_id(0)
    rs = rs_ref[0, 0]
    scale = _softplus(rs)
    bg = bg_ref[...]
    gm = gm_ref[...]
    bt = bt_ref[...]
    vm = vm_ref[...]
    wg = wg_ref[...]

    zh = _fuse_side(th_ref[...], vh_ref[...], mh_ref[...], rh_ref[...],
                    rf_ref[...], wg, bg, gm, bt, scale, vm)
    zt = _fuse_side(tt_ref[...], vt_ref[...], mt_ref[...], rt_ref[...],
                    rf_ref[...], wg, bg, gm, bt, scale, vm)
    rd = rd_ref[...]

    hr, hi = zh[:, :D // 2], zh[:, D // 2:]
    rr, ri = rd[:, :D // 2], rd[:, D // 2:]
    tr, ti = zt[:, :D // 2], zt[:, D // 2:]
    s = jnp.sum(hr * (rr * tr + ri * ti) + hi * (rr * ti - ri * tr), axis=1)

    contrib = jnp.where(
        i < POS_BLKS,
        jnp.sum(_softplus(-s)) / B_POS,
        jnp.sum(_softplus(s)) / B_NEG,
    )

    @pl.when(i == 0)
    def _():
        out_ref[...] = jnp.reshape(contrib + 1e-04 * scale * scale, (1, 1))

    @pl.when(i > 0)
    def _():
        out_ref[...] += jnp.reshape(contrib, (1, 1))


def _tc_fuse(gat_t, gat_v, gat_m2, gat_r, gat_rf, gat_rd,
             Wg, bg, gamma, beta, v_missing, rscale):
    row_spec_h = pl.BlockSpec((ROWS_B, D), lambda i: (i, 0))
    row_spec_t = pl.BlockSpec((ROWS_B, D), lambda i: (i + N_BLK, 0))
    m_spec_h = pl.BlockSpec((ROWS_B, 1), lambda i: (i, 0))
    m_spec_t = pl.BlockSpec((ROWS_B, 1), lambda i: (i + N_BLK, 0))
    rel_spec = pl.BlockSpec((ROWS_B, D), lambda i: (i, 0))

    def p_spec(shape):
        return pl.BlockSpec(shape, lambda i: (0, 0))

    return pl.pallas_call(
        _fuse_kernel,
        grid=(N_BLK,),
        in_specs=[
            row_spec_h, row_spec_t,
            row_spec_h, row_spec_t,
            m_spec_h, m_spec_t,
            row_spec_h, row_spec_t,
            rel_spec, rel_spec,
            p_spec((3 * D, D)),
            p_spec((1, D)), p_spec((1, D)), p_spec((1, D)), p_spec((1, D)),
            p_spec((1, 1)),
        ],
        out_specs=pl.BlockSpec((1, 1), lambda i: (0, 0)),
        out_shape=jax.ShapeDtypeStruct((1, 1), jnp.float32),
        compiler_params=pltpu.CompilerParams(
            dimension_semantics=("arbitrary",)),
    )(gat_t, gat_t, gat_v, gat_v, gat_m2, gat_m2, gat_r, gat_r,
      gat_rf, gat_rd, Wg, bg, gamma, beta, v_missing, rscale)


L2_ROWS = 1000
L2_BLKS = N_ENT // L2_ROWS


def _l2_kernel(er_ref, out_ref):
    i = pl.program_id(0)
    x = er_ref[...]
    part = jnp.sum(x * x) * (1e-06 / (N_ENT * D))

    @pl.when(i == 0)
    def _():
        out_ref[...] = jnp.reshape(part, (1, 1))

    @pl.when(i > 0)
    def _():
        out_ref[...] += jnp.reshape(part, (1, 1))


def _tc_l2(entity_residual):
    return pl.pallas_call(
        _l2_kernel,
        grid=(L2_BLKS,),
        in_specs=[pl.BlockSpec((L2_ROWS, D), lambda i: (i, 0))],
        out_specs=pl.BlockSpec((1, 1), lambda i: (0, 0)),
        out_shape=jax.ShapeDtypeStruct((1, 1), jnp.float32),
        compiler_params=pltpu.CompilerParams(
            dimension_semantics=("arbitrary",)),
    )(entity_residual)


def kernel(text_emb, img_emb, v_missing, entity_residual, residual_scale,
           rel_emb_fusion, Wg, bg, gamma, beta, rel_emb_dec, has_img,
           pos_triples, neg_triples):
    eids = jnp.concatenate([
        pos_triples[:, 0], neg_triples[:, 0],
        pos_triples[:, 2], neg_triples[:, 2],
    ])
    rids = jnp.concatenate([pos_triples[:, 1], neg_triples[:, 1]])
    has_img_f = has_img.astype(jnp.float32)

    gat_t, gat_v, gat_r, gat_m, gat_rf, gat_rd = _sc_gather(
        text_emb, img_emb, entity_residual, has_img_f,
        rel_emb_fusion, rel_emb_dec, eids, rids)

    bce = _tc_fuse(
        gat_t, gat_v, gat_m.reshape(N_EID, 1), gat_r, gat_rf, gat_rd,
        Wg, bg.reshape(1, D), gamma.reshape(1, D), beta.reshape(1, D),
        v_missing.reshape(1, D),
        jnp.asarray(residual_scale, jnp.float32).reshape(1, 1))

    l2 = _tc_l2(entity_residual)
    return bce[0, 0] + l2[0, 0]


# speedup vs baseline: 2.3050x; 1.2764x over previous
import functools

import jax
import jax.numpy as jnp
from jax import lax
from jax.experimental import pallas as pl
from jax.experimental.pallas import tpu as pltpu
from jax.experimental.pallas import tpu_sc as plsc

N_ENT = 100000
N_REL = 1000
D = 128
B_POS = 16384
B_NEG = 65536
B_ALL = B_POS + B_NEG
N_EID = 2 * B_ALL

NC = 2
NS = 16
NW = NC * NS
CHUNK = 128

EID_PER_W = N_EID // NW
RID_PER_W = B_ALL // NW
EID_CHUNKS = EID_PER_W // CHUNK
RID_CHUNKS = RID_PER_W // CHUNK

ROWS_B = 512
N_BLK = B_ALL // ROWS_B
POS_BLKS = B_POS // ROWS_B


def _sc_gather(text_emb, img_emb, entity_residual, has_img_f,
               rel_emb_fusion, rel_emb_dec, eids2d, rids2d):
    mesh = plsc.VectorSubcoreMesh(core_axis_name="c", subcore_axis_name="s")

    @functools.partial(
        pl.kernel,
        mesh=mesh,
        out_type=[
            jax.ShapeDtypeStruct((N_EID, D), jnp.float32),
            jax.ShapeDtypeStruct((N_EID, D), jnp.float32),
            jax.ShapeDtypeStruct((N_EID, D), jnp.float32),
            jax.ShapeDtypeStruct((N_EID,), jnp.float32),
            jax.ShapeDtypeStruct((B_ALL, D), jnp.float32),
            jax.ShapeDtypeStruct((B_ALL, D), jnp.float32),
        ],
        scratch_types=[
            pltpu.VMEM((EID_CHUNKS, CHUNK), jnp.int32),
            pltpu.VMEM((RID_CHUNKS, CHUNK), jnp.int32),
            pltpu.VMEM((2, CHUNK, D), jnp.float32),
            pltpu.VMEM((2, CHUNK, D), jnp.float32),
            pltpu.VMEM((2, CHUNK, D), jnp.float32),
            pltpu.VMEM((2, CHUNK), jnp.float32),
            pltpu.SemaphoreType.DMA,
            pltpu.SemaphoreType.DMA,
            pltpu.SemaphoreType.DMA,
            pltpu.SemaphoreType.DMA,
        ],
    )
    def k(text_h, img_h, res_h, mask_h, rf_h, rd_h, eids_h, rids_h,
          out_t, out_v, out_r, out_m, out_rf, out_rd,
          idx2d, rid2d, tb, vb, rb, mb, sg0, sg1, ss0, ss1):
        wid = lax.axis_index("c") * NS + lax.axis_index("s")
        ebase = wid * EID_PER_W
        rbase = wid * RID_PER_W
        sg = (sg0, sg1)
        ss = (ss0, ss1)

        pltpu.sync_copy(eids_h.at[wid], idx2d)
        pltpu.sync_copy(rids_h.at[wid], rid2d)

        egat = ((text_h, tb, out_t), (img_h, vb, out_v), (res_h, rb, out_r),
                (mask_h, mb, out_m))
        rgat = ((rf_h, tb, out_rf), (rd_h, vb, out_rd))

        def fire_gather(gats, idx, b):
            for src, buf, _ in gats:
                pltpu.async_copy(src.at[idx], buf.at[b], sg[b])

        def wait_gather(gats, b):
            for src, buf, _ in gats:
                pltpu.make_async_copy(src.at[pl.ds(0, CHUNK)], buf.at[b],
                                      sg[b]).wait()

        def fire_store(gats, off, b):
            for _, buf, out in gats:
                pltpu.async_copy(buf.at[b], out.at[pl.ds(off, CHUNK)], ss[b])

        def wait_store(gats, b):
            for _, buf, out in gats:
                pltpu.make_async_copy(buf.at[b], out.at[pl.ds(0, CHUNK)],
                                      ss[b]).wait()

        def run(gats, idxref, nchunks, base):
            def body(c2, _):
                for b in (0, 1):
                    c = c2 * 2 + b

                    @pl.when(c >= 2)
                    def _():
                        wait_store(gats, b)

                    fire_gather(gats, idxref.at[c], b)

                    @pl.when(c >= 1)
                    def _():
                        wait_gather(gats, 1 - b)
                        fire_store(gats, base + (c - 1) * CHUNK, 1 - b)
                return 0

            lax.fori_loop(0, nchunks // 2, body, 0)
            wait_gather(gats, 1)
            fire_store(gats, base + (nchunks - 1) * CHUNK, 1)
            wait_store(gats, 0)
            wait_store(gats, 1)

        run(egat, idx2d, EID_CHUNKS, ebase)
        run(rgat, rid2d, RID_CHUNKS, rbase)

    return k(text_emb, img_emb, entity_residual, has_img_f,
             rel_emb_fusion, rel_emb_dec, eids2d, rids2d)


def _softplus(x):
    return jnp.maximum(x, 0.0) + jnp.log1p(jnp.exp(-jnp.abs(x)))


def _fuse_side(t, v_raw, m, resid, rf, Wg, bg, gamma, beta, scale, v_missing):
    v = jnp.where(m > 0.5, v_raw, v_missing)
    x = jnp.concatenate([t, v, rf], axis=1)
    g = jax.nn.sigmoid(
        jnp.dot(x, Wg, preferred_element_type=jnp.float32) + bg)
    z = g * t + (1.0 - g) * v
    mu = jnp.mean(z, axis=-1, keepdims=True)
    zc = z - mu
    var = jnp.mean(zc * zc, axis=-1, keepdims=True)
    z = zc * jax.lax.rsqrt(var + 1e-05) * gamma + beta
    return z + scale * resid


def _fuse_kernel(th_ref, tt_ref, vh_ref, vt_ref, mh_ref, mt_ref,
                 rh_ref, rt_ref, rf_ref, rd_ref,
                 wg_ref, bg_ref, gm_ref, bt_ref, vm_ref, rs_ref,
                 out_ref):
    i = pl.program_id(0)
    rs = rs_ref[0, 0]
    scale = _softplus(rs)
    bg = bg_ref[...]
    gm = gm_ref[...]
    bt = bt_ref[...]
    vm = vm_ref[...]
    wg = wg_ref[...]

    zh = _fuse_side(th_ref[...], vh_ref[...], mh_ref[...], rh_ref[...],
                    rf_ref[...], wg, bg, gm, bt, scale, vm)
    zt = _fuse_side(tt_ref[...], vt_ref[...], mt_ref[...], rt_ref[...],
                    rf_ref[...], wg, bg, gm, bt, scale, vm)
    rd = rd_ref[...]

    hr, hi = zh[:, :D // 2], zh[:, D // 2:]
    rr, ri = rd[:, :D // 2], rd[:, D // 2:]
    tr, ti = zt[:, :D // 2], zt[:, D // 2:]
    s = jnp.sum(hr * (rr * tr + ri * ti) + hi * (rr * ti - ri * tr), axis=1)

    contrib = jnp.where(
        i < POS_BLKS,
        jnp.sum(_softplus(-s)) / B_POS,
        jnp.sum(_softplus(s)) / B_NEG,
    )

    @pl.when(i == 0)
    def _():
        out_ref[...] = jnp.reshape(contrib + 1e-04 * scale * scale, (1, 1))

    @pl.when(i > 0)
    def _():
        out_ref[...] += jnp.reshape(contrib, (1, 1))


def _tc_fuse(gat_t, gat_v, gat_m2, gat_r, gat_rf, gat_rd,
             Wg, bg, gamma, beta, v_missing, rscale):
    row_spec_h = pl.BlockSpec((ROWS_B, D), lambda i: (i, 0))
    row_spec_t = pl.BlockSpec((ROWS_B, D), lambda i: (i + N_BLK, 0))
    m_spec_h = pl.BlockSpec((ROWS_B, 1), lambda i: (i, 0))
    m_spec_t = pl.BlockSpec((ROWS_B, 1), lambda i: (i + N_BLK, 0))
    rel_spec = pl.BlockSpec((ROWS_B, D), lambda i: (i, 0))

    def p_spec(shape):
        return pl.BlockSpec(shape, lambda i: (0, 0))

    return pl.pallas_call(
        _fuse_kernel,
        grid=(N_BLK,),
        in_specs=[
            row_spec_h, row_spec_t,
            row_spec_h, row_spec_t,
            m_spec_h, m_spec_t,
            row_spec_h, row_spec_t,
            rel_spec, rel_spec,
            p_spec((3 * D, D)),
            p_spec((1, D)), p_spec((1, D)), p_spec((1, D)), p_spec((1, D)),
            p_spec((1, 1)),
        ],
        out_specs=pl.BlockSpec((1, 1), lambda i: (0, 0)),
        out_shape=jax.ShapeDtypeStruct((1, 1), jnp.float32),
        compiler_params=pltpu.CompilerParams(
            dimension_semantics=("arbitrary",)),
    )(gat_t, gat_t, gat_v, gat_v, gat_m2, gat_m2, gat_r, gat_r,
      gat_rf, gat_rd, Wg, bg, gamma, beta, v_missing, rscale)


L2_ROWS = 1000
L2_BLKS = N_ENT // L2_ROWS


def _l2_kernel(er_ref, out_ref):
    i = pl.program_id(0)
    x = er_ref[...]
    part = jnp.sum(x * x) * (1e-06 / (N_ENT * D))

    @pl.when(i == 0)
    def _():
        out_ref[...] = jnp.reshape(part, (1, 1))

    @pl.when(i > 0)
    def _():
        out_ref[...] += jnp.reshape(part, (1, 1))


def _tc_l2(entity_residual):
    return pl.pallas_call(
        _l2_kernel,
        grid=(L2_BLKS,),
        in_specs=[pl.BlockSpec((L2_ROWS, D), lambda i: (i, 0))],
        out_specs=pl.BlockSpec((1, 1), lambda i: (0, 0)),
        out_shape=jax.ShapeDtypeStruct((1, 1), jnp.float32),
        compiler_params=pltpu.CompilerParams(
            dimension_semantics=("arbitrary",)),
    )(entity_residual)


def kernel(text_emb, img_emb, v_missing, entity_residual, residual_scale,
           rel_emb_fusion, Wg, bg, gamma, beta, rel_emb_dec, has_img,
           pos_triples, neg_triples):
    eids = jnp.concatenate([
        pos_triples[:, 0], neg_triples[:, 0],
        pos_triples[:, 2], neg_triples[:, 2],
    ])
    rids = jnp.concatenate([pos_triples[:, 1], neg_triples[:, 1]])
    has_img_f = has_img.astype(jnp.float32)

    gat_t, gat_v, gat_r, gat_m, gat_rf, gat_rd = _sc_gather(
        text_emb, img_emb, entity_residual, has_img_f,
        rel_emb_fusion, rel_emb_dec,
        eids.reshape(NW, EID_CHUNKS, CHUNK),
        rids.reshape(NW, RID_CHUNKS, CHUNK))

    bce = _tc_fuse(
        gat_t, gat_v, gat_m.reshape(N_EID, 1), gat_r, gat_rf, gat_rd,
        Wg, bg.reshape(1, D), gamma.reshape(1, D), beta.reshape(1, D),
        v_missing.reshape(1, D),
        jnp.asarray(residual_scale, jnp.float32).reshape(1, 1))

    l2 = _tc_l2(entity_residual)
    return bce[0, 0] + l2[0, 0]
